# baseline (speedup 1.0000x reference)
import functools

import jax
import jax.numpy as jnp
from jax import lax
from jax.experimental import pallas as pl
from jax.experimental.pallas import tpu as pltpu
from jax.experimental.pallas import tpu_sc as plsc

NN = 10000
EE = 320000
DD = 128
HH = 16
ALPHA_NEG = 0.2

NC = 2
NS = 16
NWK = NC * NS
EP = EE // NWK
C2 = 80
NCH = EP // C2
EPB = EE // NS
C2B = 400
NCHB = EPB // C2B
CH = 1000

_f32 = jnp.float32
_i32 = jnp.int32

_MESH = plsc.VectorSubcoreMesh(core_axis_name="c", subcore_axis_name="s")


def _k1_body(x_ref, deg_ref, w2t_ref, b2_ref, wat_ref, wbt_ref, bf1_ref,
             u_ref, v_ref, xdh_ref):
    x = x_ref[...]
    nh = jnp.dot(x, w2t_ref[...], preferred_element_type=_f32) + b2_ref[...]
    a = jnp.dot(nh, wat_ref[...], preferred_element_type=_f32) + bf1_ref[...]
    b = jnp.dot(nh, wbt_ref[...], preferred_element_type=_f32)
    u_ref[...] = jnp.concatenate([nh, a], axis=1)
    v_ref[...] = jnp.concatenate([nh, b], axis=1)
    xd = x * deg_ref[...]
    xdh_ref[0] = xd[:, :DD // 2]
    xdh_ref[1] = xd[:, DD // 2:]


def _k1(x, deg2, w2t, b2, wat, wbt, bf1r):
    bn = 1000
    grid = NN // bn
    return pl.pallas_call(
        _k1_body,
        grid=(grid,),
        in_specs=[
            pl.BlockSpec((bn, DD), lambda i: (i, 0)),
            pl.BlockSpec((bn, 1), lambda i: (i, 0)),
            pl.BlockSpec((DD, HH), lambda i: (0, 0)),
            pl.BlockSpec((1, HH), lambda i: (0, 0)),
            pl.BlockSpec((HH, HH), lambda i: (0, 0)),
            pl.BlockSpec((HH, HH), lambda i: (0, 0)),
            pl.BlockSpec((1, HH), lambda i: (0, 0)),
        ],
        out_specs=[
            pl.BlockSpec((bn, 2 * HH), lambda i: (i, 0)),
            pl.BlockSpec((bn, 2 * HH), lambda i: (i, 0)),
            pl.BlockSpec((NC, bn, DD // 2), lambda i: (0, i, 0)),
        ],
        out_shape=[
            jax.ShapeDtypeStruct((NN, 2 * HH), _f32),
            jax.ShapeDtypeStruct((NN, 2 * HH), _f32),
            jax.ShapeDtypeStruct((NC, NN, DD // 2), _f32),
        ],
    )(x, deg2, w2t, b2, wat, wbt, bf1r)


@functools.partial(
    pl.kernel,
    out_type=(
        jax.ShapeDtypeStruct((EE, 2 * HH), _f32),
        jax.ShapeDtypeStruct((EE, 2 * HH), _f32),
    ),
    mesh=_MESH,
    compiler_params=pltpu.CompilerParams(use_tc_tiling_on_sc=False, needs_layout_passes=False, skip_device_barrier=True),
    scratch_types=[
        pltpu.VMEM((EP,), _i32),
        pltpu.VMEM((EP,), _i32),
        pltpu.VMEM((CH, 2 * HH), _f32),
        pltpu.VMEM((CH, 2 * HH), _f32),
        pltpu.SemaphoreType.DMA,
        pltpu.SemaphoreType.DMA,
    ],
)
def _k2(u_hbm, v_hbm, row_hbm, col_hbm, hsrc_hbm, hdst_hbm,
        rows_v, cols_v, ubuf, vbuf, sem1, sem2):
    c = lax.axis_index("c")
    s = lax.axis_index("s")
    w = s * NC + c
    base = w * EP
    pltpu.sync_copy(row_hbm.at[pl.ds(base, EP)], rows_v)
    pltpu.sync_copy(col_hbm.at[pl.ds(base, EP)], cols_v)

    def body(ci, carry):
        e0 = ci * CH
        cp1 = pltpu.async_copy(u_hbm.at[rows_v.at[pl.ds(e0, CH)]], ubuf, sem1)
        cp2 = pltpu.async_copy(v_hbm.at[cols_v.at[pl.ds(e0, CH)]], vbuf, sem2)
        cp1.wait()
        cp2.wait()
        pltpu.sync_copy(ubuf, hsrc_hbm.at[pl.ds(base + e0, CH)])
        pltpu.sync_copy(vbuf, hdst_hbm.at[pl.ds(base + e0, CH)])
        return carry

    lax.fori_loop(0, EP // CH, body, 0)


def _k3_body(hs_ref, hd_ref, wct_ref, wf2t_ref, bf2_ref, csh_ref,
             s0_ref, s1_ref):
    hs = hs_ref[...]
    hd = hd_ref[...]
    nh_s = hs[:, :HH]
    nh_d = hd[:, :HH]
    d = jnp.abs(nh_d - nh_s)
    z = hs[:, HH:] + hd[:, HH:] + jnp.dot(d, wct_ref[...],
                                          preferred_element_type=_f32)
    za = jnp.where(z >= 0, z, ALPHA_NEG * z)
    t0 = jnp.dot(za, wf2t_ref[...], preferred_element_type=_f32) + bf2_ref[...]
    s0_ref[...] = jax.nn.sigmoid(t0)
    zb = z + csh_ref[...]
    zba = jnp.where(zb >= 0, zb, ALPHA_NEG * zb)
    t1 = jnp.dot(zba, wf2t_ref[...], preferred_element_type=_f32) + bf2_ref[...]
    s1_ref[...] = jax.nn.sigmoid(t1)


def _k3(hsrc, hdst, wct, wf2t, bf2r, cshift):
    be = 8000
    grid = EE // be
    return pl.pallas_call(
        _k3_body,
        grid=(grid,),
        in_specs=[
            pl.BlockSpec((be, 2 * HH), lambda i: (i, 0)),
            pl.BlockSpec((be, 2 * HH), lambda i: (i, 0)),
            pl.BlockSpec((HH, HH), lambda i: (0, 0)),
            pl.BlockSpec((HH, 1), lambda i: (0, 0)),
            pl.BlockSpec((1, 1), lambda i: (0, 0)),
            pl.BlockSpec((1, HH), lambda i: (0, 0)),
        ],
        out_specs=[
            pl.BlockSpec((be, 1), lambda i: (i, 0)),
            pl.BlockSpec((be, 1), lambda i: (i, 0)),
        ],
        out_shape=[
            jax.ShapeDtypeStruct((EE, 1), _f32),
            jax.ShapeDtypeStruct((EE, 1), _f32),
        ],
    )(hsrc, hdst, wct, wf2t, bf2r, cshift)


NP = 10240
STR = NP // NS
P1C = 2000


@functools.partial(
    pl.kernel,
    out_type=(
        jax.ShapeDtypeStruct((EE,), _f32),
        jax.ShapeDtypeStruct((NWK, NN), _f32),
        jax.ShapeDtypeStruct((NC, NS, NP), _f32),
        jax.ShapeDtypeStruct((NC, NP), _f32),
    ),
    mesh=_MESH,
    compiler_params=pltpu.CompilerParams(use_tc_tiling_on_sc=False, needs_layout_passes=False, skip_device_barrier=True),
    scratch_types=[
        pltpu.VMEM((NP,), _f32),
        pltpu.VMEM((P1C,), _i32),
        pltpu.VMEM((P1C,), _f32),
        pltpu.VMEM((STR,), _f32),
        pltpu.VMEM((STR,), _f32),
        pltpu.VMEM((NP,), _f32),
        pltpu.VMEM((EP,), _i32),
        pltpu.VMEM((EP,), _i32),
        pltpu.VMEM((EP,), _f32),
        pltpu.VMEM((EP,), _f32),
    ],
)
def _k456(row_hbm, col_hbm, s0_hbm, s1_hbm, adjp_hbm,
          ef_hbm, efp_hbm, part_hbm, mean_hbm,
          acc, rows_c, s0_c, stripe, tmp, mean0_v, rows1, cols1, s1_v, ef_v):
    c = lax.axis_index("c")
    s = lax.axis_index("s")
    w = s * NC + c

    def zero(g, carry):
        acc[pl.ds(g * 16, 16)] = jnp.zeros((16,), _f32)
        return carry

    lax.fori_loop(0, NP // 16, zero, 0)

    for cb in range(EPB // P1C):
        eb = s * EPB + cb * P1C
        pltpu.sync_copy(row_hbm.at[pl.ds(eb, P1C)], rows_c)
        pltpu.sync_copy(s0_hbm.at[pl.ds(eb, P1C)], s0_c)

        def p1(g, carry):
            o = g * 16
            plsc.addupdate_scatter(acc, [rows_c[pl.ds(o, 16)]],
                                   s0_c[pl.ds(o, 16)])
            return carry

        lax.fori_loop(0, P1C // 16, p1, 0)
    pltpu.sync_copy(acc, part_hbm.at[c, s])
    plsc.subcore_barrier()

    def zstr(g, carry):
        stripe[pl.ds(g * 16, 16)] = jnp.zeros((16,), _f32)
        return carry

    lax.fori_loop(0, STR // 16, zstr, 0)
    for t in range(NS):
        pltpu.sync_copy(part_hbm.at[c, t, pl.ds(s * STR, STR)], tmp)

        def radd(g, carry):
            o = g * 16
            stripe[pl.ds(o, 16)] = stripe[pl.ds(o, 16)] + tmp[pl.ds(o, 16)]
            return carry

        lax.fori_loop(0, STR // 16, radd, 0)
    pltpu.sync_copy(adjp_hbm.at[pl.ds(s * STR, STR)], tmp)

    def rdiv(g, carry):
        o = g * 16
        stripe[pl.ds(o, 16)] = stripe[pl.ds(o, 16)] / tmp[pl.ds(o, 16)]
        return carry

    lax.fori_loop(0, STR // 16, rdiv, 0)
    pltpu.sync_copy(stripe, mean_hbm.at[c, pl.ds(s * STR, STR)])
    plsc.subcore_barrier()

    pltpu.sync_copy(mean_hbm.at[c], mean0_v)
    pltpu.sync_copy(row_hbm.at[pl.ds(w * EP, EP)], rows1)
    pltpu.sync_copy(col_hbm.at[pl.ds(w * EP, EP)], cols1)
    pltpu.sync_copy(s1_hbm.at[pl.ds(w * EP, EP)], s1_v)
    lax.fori_loop(0, NP // 16, zero, 0)

    def flat(i, carry):
        o = i * 16
        r16 = rows1[pl.ds(o, 16)]
        c16 = cols1[pl.ds(o, 16)]
        mr = plsc.load_gather(mean0_v, [r16])
        mc = plsc.load_gather(mean0_v, [c16])
        ef16 = mr * mc * s1_v[pl.ds(o, 16)]
        ef_v[pl.ds(o, 16)] = ef16
        plsc.addupdate_scatter(acc, [r16], ef16)
        return carry

    lax.fori_loop(0, EP // 16, flat, 0)
    pltpu.sync_copy(ef_v, ef_hbm.at[pl.ds(w * EP, EP)])
    pltpu.sync_copy(acc.at[pl.ds(0, NN)], efp_hbm.at[w])


@functools.partial(
    pl.kernel,
    out_type=jax.ShapeDtypeStruct((NC, NN, DD // 2), _f32),
    mesh=_MESH,
    compiler_params=pltpu.CompilerParams(use_tc_tiling_on_sc=False, needs_layout_passes=False, skip_device_barrier=True),
    scratch_types=[
        pltpu.VMEM((NCHB, C2B), _i32),
        pltpu.VMEM((EPB,), _i32),
        pltpu.VMEM((C2B,), _f32),
        pltpu.VMEM((C2B, DD // 2), _f32),
        pltpu.VMEM_SHARED((NN, DD // 2), _f32),
        pltpu.SemaphoreType.DMA,
    ],
)
def _k6b(row3b_hbm, col_hbm, ef_hbm, xdh_hbm, zeros_hbm, aggp_hbm,
         rows2, cols1, efc, rowbuf, agg_sh, sem):
    c = lax.axis_index("c")
    s = lax.axis_index("s")
    pltpu.sync_copy(row3b_hbm.at[s], rows2)
    pltpu.sync_copy(col_hbm.at[pl.ds(s * EPB, EPB)], cols1)

    @pl.when(s == 0)
    def _zero_agg():
        pltpu.sync_copy(zeros_hbm, agg_sh)

    plsc.subcore_barrier()

    def chunk(ci, carry):
        base = s * EPB + ci * C2B
        cp = pltpu.async_copy(
            xdh_hbm.at[c].at[cols1.at[pl.ds(ci * C2B, C2B)]], rowbuf, sem)
        pltpu.sync_copy(ef_hbm.at[pl.ds(base, C2B)], efc)
        cp.wait()
        for g in range(C2B // 16):
            ef16 = efc[pl.ds(g * 16, 16)]
            for j in range(16):
                e = ef16[j]
                bj = g * 16 + j
                for r in range(DD // 32):
                    sl = pl.ds(r * 16, 16)
                    rowbuf[bj, sl] = rowbuf[bj, sl] * e
        pltpu.sync_copy(rowbuf, agg_sh.at[rows2.at[ci]], add=True)
        return carry

    lax.fori_loop(0, NCHB, chunk, 0)
    plsc.subcore_barrier()

    @pl.when(s == 0)
    def _dump_agg():
        pltpu.sync_copy(agg_sh, aggp_hbm.at[c])


def _k7_body(aggp_ref, efp_ref, deg_ref, adj_ref, x_ref, out_ref):
    agg = jnp.concatenate([aggp_ref[0], aggp_ref[1]], axis=1)
    efsum = jnp.sum(efp_ref[...], axis=1)[:, None]
    out_ref[...] = (agg * deg_ref[...]
                    + (1.0 - efsum / adj_ref[...]) * x_ref[...])


def _k7(aggp, efp, deg2, adj2, x):
    bn = 1000
    grid = NN // bn
    return pl.pallas_call(
        _k7_body,
        grid=(grid,),
        in_specs=[
            pl.BlockSpec((NC, bn, DD // 2), lambda i: (0, i, 0)),
            pl.BlockSpec((bn, NWK), lambda i: (i, 0)),
            pl.BlockSpec((bn, 1), lambda i: (i, 0)),
            pl.BlockSpec((bn, 1), lambda i: (i, 0)),
            pl.BlockSpec((bn, DD), lambda i: (i, 0)),
        ],
        out_specs=pl.BlockSpec((bn, DD), lambda i: (i, 0)),
        out_shape=jax.ShapeDtypeStruct((NN, DD), _f32),
    )(aggp, efp, deg2, adj2, x)


def kernel(input, adj, edge_factor, edges, adj_sparse_sum_rowwise, degree,
           iftrain, W2mini, b2mini, Wf1, bf1, Wf2, bf2, attention_bias):
    x = input
    row = edges[0]
    col = edges[1]
    w2t = W2mini.T
    b2r = b2mini.reshape(1, HH)
    wat = Wf1[:, :HH].T
    wbt = Wf1[:, HH:2 * HH].T
    wct = Wf1[:, 2 * HH:].T
    bf1r = bf1.reshape(1, HH)
    wf2t = Wf2.T
    bf2r = bf2.reshape(1, 1)
    cshift = attention_bias @ wat + attention_bias @ wbt
    deg2 = degree.reshape(NN, 1)
    adj2 = adj_sparse_sum_rowwise.reshape(NN, 1)
    adjp = jnp.concatenate(
        [adj_sparse_sum_rowwise.reshape(NN), jnp.ones((NP - NN,), _f32)])
    row3b = row.reshape(NS, NCHB, C2B)
    zeros_nd = jnp.zeros((NN, DD // 2), _f32)

    u, v, xdh = _k1(x, deg2, w2t, b2r, wat, wbt, bf1r)
    hsrc, hdst = _k2(u, v, row, col)
    s0, s1 = _k3(hsrc, hdst, wct, wf2t, bf2r, cshift)
    ef, efp, _p, _m = _k456(row, col, s0.reshape(EE), s1.reshape(EE), adjp)
    aggp = _k6b(row3b, col, ef, xdh, zeros_nd)
    final_h = _k7(aggp, efp.T, deg2, adj2, x)
    return (final_h, ef)


# speedup vs baseline: 1.2687x; 1.2687x over previous
import functools

import jax
import jax.numpy as jnp
from jax import lax
from jax.experimental import pallas as pl
from jax.experimental.pallas import tpu as pltpu
from jax.experimental.pallas import tpu_sc as plsc

NN = 10000
EE = 320000
DD = 128
HH = 16
ALPHA_NEG = 0.2

NC = 2
NS = 16
NWK = NC * NS
EP = EE // NWK
C2 = 80
NCH = EP // C2
EPB = EE // NS
C2B = 400
NCHB = EPB // C2B
CH = 1000

_f32 = jnp.float32
_i32 = jnp.int32

_MESH = plsc.VectorSubcoreMesh(core_axis_name="c", subcore_axis_name="s")


def _k1_body(x_ref, deg_ref, w2t_ref, b2_ref, wat_ref, wbt_ref, bf1_ref,
             u_ref, v_ref, xdh_ref):
    x = x_ref[...]
    nh = jnp.dot(x, w2t_ref[...], preferred_element_type=_f32) + b2_ref[...]
    a = jnp.dot(nh, wat_ref[...], preferred_element_type=_f32) + bf1_ref[...]
    b = jnp.dot(nh, wbt_ref[...], preferred_element_type=_f32)
    u_ref[...] = jnp.concatenate([nh, a], axis=1)
    v_ref[...] = jnp.concatenate([nh, b], axis=1)
    xd = x * deg_ref[...]
    xdh_ref[0] = xd[:, :DD // 2]
    xdh_ref[1] = xd[:, DD // 2:]


def _k1(x, deg2, w2t, b2, wat, wbt, bf1r):
    bn = 1000
    grid = NN // bn
    return pl.pallas_call(
        _k1_body,
        grid=(grid,),
        in_specs=[
            pl.BlockSpec((bn, DD), lambda i: (i, 0)),
            pl.BlockSpec((bn, 1), lambda i: (i, 0)),
            pl.BlockSpec((DD, HH), lambda i: (0, 0)),
            pl.BlockSpec((1, HH), lambda i: (0, 0)),
            pl.BlockSpec((HH, HH), lambda i: (0, 0)),
            pl.BlockSpec((HH, HH), lambda i: (0, 0)),
            pl.BlockSpec((1, HH), lambda i: (0, 0)),
        ],
        out_specs=[
            pl.BlockSpec((bn, 2 * HH), lambda i: (i, 0)),
            pl.BlockSpec((bn, 2 * HH), lambda i: (i, 0)),
            pl.BlockSpec((NC, bn, DD // 2), lambda i: (0, i, 0)),
        ],
        out_shape=[
            jax.ShapeDtypeStruct((NN, 2 * HH), _f32),
            jax.ShapeDtypeStruct((NN, 2 * HH), _f32),
            jax.ShapeDtypeStruct((NC, NN, DD // 2), _f32),
        ],
    )(x, deg2, w2t, b2, wat, wbt, bf1r)


@functools.partial(
    pl.kernel,
    out_type=(
        jax.ShapeDtypeStruct((EE, 2 * HH), _f32),
        jax.ShapeDtypeStruct((EE, 2 * HH), _f32),
    ),
    mesh=_MESH,
    compiler_params=pltpu.CompilerParams(use_tc_tiling_on_sc=False, needs_layout_passes=False, skip_device_barrier=True),
    scratch_types=[
        pltpu.VMEM((EP,), _i32),
        pltpu.VMEM((EP,), _i32),
        pltpu.VMEM((CH, 2 * HH), _f32),
        pltpu.VMEM((CH, 2 * HH), _f32),
        pltpu.SemaphoreType.DMA,
        pltpu.SemaphoreType.DMA,
    ],
)
def _k2(u_hbm, v_hbm, row_hbm, col_hbm, hsrc_hbm, hdst_hbm,
        rows_v, cols_v, ubuf, vbuf, sem1, sem2):
    c = lax.axis_index("c")
    s = lax.axis_index("s")
    w = s * NC + c
    base = w * EP
    pltpu.sync_copy(row_hbm.at[pl.ds(base, EP)], rows_v)
    pltpu.sync_copy(col_hbm.at[pl.ds(base, EP)], cols_v)

    def body(ci, carry):
        e0 = ci * CH
        cp1 = pltpu.async_copy(u_hbm.at[rows_v.at[pl.ds(e0, CH)]], ubuf, sem1)
        cp2 = pltpu.async_copy(v_hbm.at[cols_v.at[pl.ds(e0, CH)]], vbuf, sem2)
        cp1.wait()
        cp2.wait()
        pltpu.sync_copy(ubuf, hsrc_hbm.at[pl.ds(base + e0, CH)])
        pltpu.sync_copy(vbuf, hdst_hbm.at[pl.ds(base + e0, CH)])
        return carry

    lax.fori_loop(0, EP // CH, body, 0)


def _k3_body(hs_ref, hd_ref, wct_ref, wf2t_ref, bf2_ref, csh_ref,
             s0_ref, s1_ref):
    hs = hs_ref[...]
    hd = hd_ref[...]
    s0cols = []
    s1cols = []
    for g in range(4):
        hsg = hs[:, g * 32:(g + 1) * 32]
        hdg = hd[:, g * 32:(g + 1) * 32]
        d = jnp.abs(hdg[:, :HH] - hsg[:, :HH])
        z = hsg[:, HH:] + hdg[:, HH:] + jnp.dot(
            d, wct_ref[...], preferred_element_type=_f32)
        za = jnp.where(z >= 0, z, ALPHA_NEG * z)
        t0 = jnp.dot(za, wf2t_ref[...], preferred_element_type=_f32)
        s0cols.append(jax.nn.sigmoid(t0 + bf2_ref[...]))
        zb = z + csh_ref[...]
        zba = jnp.where(zb >= 0, zb, ALPHA_NEG * zb)
        t1 = jnp.dot(zba, wf2t_ref[...], preferred_element_type=_f32)
        s1cols.append(jax.nn.sigmoid(t1 + bf2_ref[...]))
    s0_ref[...] = jnp.concatenate(s0cols, axis=1)
    s1_ref[...] = jnp.concatenate(s1cols, axis=1)


def _k3(hsrcp, hdstp, wct, wf2t, bf2r, cshift):
    bq = 2000
    grid = (EE // 4) // bq
    return pl.pallas_call(
        _k3_body,
        grid=(grid,),
        in_specs=[
            pl.BlockSpec((bq, DD), lambda i: (i, 0)),
            pl.BlockSpec((bq, DD), lambda i: (i, 0)),
            pl.BlockSpec((HH, HH), lambda i: (0, 0)),
            pl.BlockSpec((HH, 1), lambda i: (0, 0)),
            pl.BlockSpec((1, 1), lambda i: (0, 0)),
            pl.BlockSpec((1, HH), lambda i: (0, 0)),
        ],
        out_specs=[
            pl.BlockSpec((bq, 4), lambda i: (i, 0)),
            pl.BlockSpec((bq, 4), lambda i: (i, 0)),
        ],
        out_shape=[
            jax.ShapeDtypeStruct((EE // 4, 4), _f32),
            jax.ShapeDtypeStruct((EE // 4, 4), _f32),
        ],
    )(hsrcp, hdstp, wct, wf2t, bf2r, cshift)


NP = 10240
STR = NP // NS
P1C = 2000


@functools.partial(
    pl.kernel,
    out_type=(
        jax.ShapeDtypeStruct((EE,), _f32),
        jax.ShapeDtypeStruct((NWK, NN), _f32),
        jax.ShapeDtypeStruct((NC, NS, NP), _f32),
        jax.ShapeDtypeStruct((NC, NP), _f32),
    ),
    mesh=_MESH,
    compiler_params=pltpu.CompilerParams(use_tc_tiling_on_sc=False, needs_layout_passes=False, skip_device_barrier=True),
    scratch_types=[
        pltpu.VMEM((NP,), _f32),
        pltpu.VMEM((P1C,), _i32),
        pltpu.VMEM((P1C,), _f32),
        pltpu.VMEM((STR,), _f32),
        pltpu.VMEM((STR,), _f32),
        pltpu.VMEM((NP,), _f32),
        pltpu.VMEM((EP,), _i32),
        pltpu.VMEM((EP,), _i32),
        pltpu.VMEM((EP,), _f32),
        pltpu.VMEM((EP,), _f32),
    ],
)
def _k456(row_hbm, col_hbm, s0_hbm, s1_hbm, adjp_hbm,
          ef_hbm, efp_hbm, part_hbm, mean_hbm,
          acc, rows_c, s0_c, stripe, tmp, mean0_v, rows1, cols1, s1_v, ef_v):
    c = lax.axis_index("c")
    s = lax.axis_index("s")
    w = s * NC + c

    def zero(g, carry):
        acc[pl.ds(g * 16, 16)] = jnp.zeros((16,), _f32)
        return carry

    lax.fori_loop(0, NP // 16, zero, 0)

    for cb in range(EPB // P1C):
        eb = s * EPB + cb * P1C
        pltpu.sync_copy(row_hbm.at[pl.ds(eb, P1C)], rows_c)
        pltpu.sync_copy(s0_hbm.at[pl.ds(eb, P1C)], s0_c)

        def p1(g, carry):
            o = g * 16
            plsc.addupdate_scatter(acc, [rows_c[pl.ds(o, 16)]],
                                   s0_c[pl.ds(o, 16)])
            return carry

        lax.fori_loop(0, P1C // 16, p1, 0)
    pltpu.sync_copy(acc, part_hbm.at[c, s])
    plsc.subcore_barrier()

    def zstr(g, carry):
        stripe[pl.ds(g * 16, 16)] = jnp.zeros((16,), _f32)
        return carry

    lax.fori_loop(0, STR // 16, zstr, 0)
    for t in range(NS):
        pltpu.sync_copy(part_hbm.at[c, t, pl.ds(s * STR, STR)], tmp)

        def radd(g, carry):
            o = g * 16
            stripe[pl.ds(o, 16)] = stripe[pl.ds(o, 16)] + tmp[pl.ds(o, 16)]
            return carry

        lax.fori_loop(0, STR // 16, radd, 0)
    pltpu.sync_copy(adjp_hbm.at[pl.ds(s * STR, STR)], tmp)

    def rdiv(g, carry):
        o = g * 16
        stripe[pl.ds(o, 16)] = stripe[pl.ds(o, 16)] / tmp[pl.ds(o, 16)]
        return carry

    lax.fori_loop(0, STR // 16, rdiv, 0)
    pltpu.sync_copy(stripe, mean_hbm.at[c, pl.ds(s * STR, STR)])
    plsc.subcore_barrier()

    pltpu.sync_copy(mean_hbm.at[c], mean0_v)
    pltpu.sync_copy(row_hbm.at[pl.ds(w * EP, EP)], rows1)
    pltpu.sync_copy(col_hbm.at[pl.ds(w * EP, EP)], cols1)
    pltpu.sync_copy(s1_hbm.at[pl.ds(w * EP, EP)], s1_v)
    lax.fori_loop(0, NP // 16, zero, 0)

    def flat(i, carry):
        o = i * 16
        r16 = rows1[pl.ds(o, 16)]
        c16 = cols1[pl.ds(o, 16)]
        mr = plsc.load_gather(mean0_v, [r16])
        mc = plsc.load_gather(mean0_v, [c16])
        ef16 = mr * mc * s1_v[pl.ds(o, 16)]
        ef_v[pl.ds(o, 16)] = ef16
        plsc.addupdate_scatter(acc, [r16], ef16)
        return carry

    lax.fori_loop(0, EP // 16, flat, 0)
    pltpu.sync_copy(ef_v, ef_hbm.at[pl.ds(w * EP, EP)])
    pltpu.sync_copy(acc.at[pl.ds(0, NN)], efp_hbm.at[w])


@functools.partial(
    pl.kernel,
    out_type=jax.ShapeDtypeStruct((NC, NN, DD // 2), _f32),
    mesh=_MESH,
    compiler_params=pltpu.CompilerParams(use_tc_tiling_on_sc=False, needs_layout_passes=False, skip_device_barrier=True),
    scratch_types=[
        pltpu.VMEM((NCHB, C2B), _i32),
        pltpu.VMEM((EPB,), _i32),
        pltpu.VMEM((C2B,), _f32),
        pltpu.VMEM((C2B, DD // 2), _f32),
        pltpu.VMEM_SHARED((NN, DD // 2), _f32),
        pltpu.SemaphoreType.DMA,
    ],
)
def _k6b(row3b_hbm, col_hbm, ef_hbm, xdh_hbm, zeros_hbm, aggp_hbm,
         rows2, cols1, efc, rowbuf, agg_sh, sem):
    c = lax.axis_index("c")
    s = lax.axis_index("s")
    pltpu.sync_copy(row3b_hbm.at[s], rows2)
    pltpu.sync_copy(col_hbm.at[pl.ds(s * EPB, EPB)], cols1)

    @pl.when(s == 0)
    def _zero_agg():
        pltpu.sync_copy(zeros_hbm, agg_sh)

    plsc.subcore_barrier()

    def chunk(ci, carry):
        base = s * EPB + ci * C2B
        cp = pltpu.async_copy(
            xdh_hbm.at[c].at[cols1.at[pl.ds(ci * C2B, C2B)]], rowbuf, sem)
        pltpu.sync_copy(ef_hbm.at[pl.ds(base, C2B)], efc)
        cp.wait()
        for g in range(C2B // 16):
            ef16 = efc[pl.ds(g * 16, 16)]
            for j in range(16):
                e = ef16[j]
                bj = g * 16 + j
                for r in range(DD // 32):
                    sl = pl.ds(r * 16, 16)
                    rowbuf[bj, sl] = rowbuf[bj, sl] * e
        pltpu.sync_copy(rowbuf, agg_sh.at[rows2.at[ci]], add=True)
        return carry

    lax.fori_loop(0, NCHB, chunk, 0)
    plsc.subcore_barrier()

    @pl.when(s == 0)
    def _dump_agg():
        pltpu.sync_copy(agg_sh, aggp_hbm.at[c])


def _k7_body(aggp_ref, efp_ref, deg_ref, adj_ref, x_ref, out_ref):
    agg = jnp.concatenate([aggp_ref[0], aggp_ref[1]], axis=1)
    efsum = jnp.sum(efp_ref[...], axis=1)[:, None]
    out_ref[...] = (agg * deg_ref[...]
                    + (1.0 - efsum / adj_ref[...]) * x_ref[...])


def _k7(aggp, efp, deg2, adj2, x):
    bn = 1000
    grid = NN // bn
    return pl.pallas_call(
        _k7_body,
        grid=(grid,),
        in_specs=[
            pl.BlockSpec((NC, bn, DD // 2), lambda i: (0, i, 0)),
            pl.BlockSpec((bn, NWK), lambda i: (i, 0)),
            pl.BlockSpec((bn, 1), lambda i: (i, 0)),
            pl.BlockSpec((bn, 1), lambda i: (i, 0)),
            pl.BlockSpec((bn, DD), lambda i: (i, 0)),
        ],
        out_specs=pl.BlockSpec((bn, DD), lambda i: (i, 0)),
        out_shape=jax.ShapeDtypeStruct((NN, DD), _f32),
    )(aggp, efp, deg2, adj2, x)


def kernel(input, adj, edge_factor, edges, adj_sparse_sum_rowwise, degree,
           iftrain, W2mini, b2mini, Wf1, bf1, Wf2, bf2, attention_bias):
    x = input
    row = edges[0]
    col = edges[1]
    w2t = W2mini.T
    b2r = b2mini.reshape(1, HH)
    wat = Wf1[:, :HH].T
    wbt = Wf1[:, HH:2 * HH].T
    wct = Wf1[:, 2 * HH:].T
    bf1r = bf1.reshape(1, HH)
    wf2t = Wf2.T
    bf2r = bf2.reshape(1, 1)
    cshift = attention_bias @ wat + attention_bias @ wbt
    deg2 = degree.reshape(NN, 1)
    adj2 = adj_sparse_sum_rowwise.reshape(NN, 1)
    adjp = jnp.concatenate(
        [adj_sparse_sum_rowwise.reshape(NN), jnp.ones((NP - NN,), _f32)])
    row3b = row.reshape(NS, NCHB, C2B)
    zeros_nd = jnp.zeros((NN, DD // 2), _f32)

    u, v, xdh = _k1(x, deg2, w2t, b2r, wat, wbt, bf1r)
    hsrc, hdst = _k2(u, v, row, col)
    s0, s1 = _k3(hsrc.reshape(EE // 4, DD), hdst.reshape(EE // 4, DD),
                 wct, wf2t, bf2r, cshift)
    ef, efp, _p, _m = _k456(row, col, s0.reshape(EE), s1.reshape(EE), adjp)
    aggp = _k6b(row3b, col, ef, xdh, zeros_nd)
    final_h = _k7(aggp, efp.T, deg2, adj2, x)
    return (final_h, ef)
